# baseline (speedup 1.0000x reference)
import functools

import jax
import jax.numpy as jnp
import numpy as np
from jax import lax
from jax.experimental import pallas as pl
from jax.experimental.pallas import tpu as pltpu
from jax.experimental.pallas import tpu_sc as plsc

K_POINTS = 256
H = W = 64
C = 32
CP = 128
NUM_WORKERS = 32
KP_PER_WORKER = K_POINTS // NUM_WORKERS

_ii = np.arange(5)
_cheb = np.maximum(np.abs(_ii - 2)[:, None], np.abs(_ii - 2)[None, :]).astype(np.float32)
_wmap = (np.float32(0.5) + np.float32(0.1) * _cheb).astype(np.float32)
_W2 = [float(np.float32(w) * np.float32(w)) for w in _wmap.reshape(-1)]


def _enc_body(q_ref, w_ref, b_ref, o_ref):
    q = q_ref[...]
    p = q.reshape(3, H, 4, W, 4).transpose(1, 3, 0, 2, 4).reshape(H * W, 48)
    y = jnp.dot(p, w_ref[...], preferred_element_type=jnp.float32)
    o_ref[...] = jnp.maximum(y + b_ref[...], 0.0)


def _encoder_keymap(query, W1, b1):
    patches = query[0].reshape(3 * 256, 256)
    wf = jnp.zeros((48, CP), jnp.float32).at[:, :C].set(W1.reshape(C, 48).T)
    b2d = jnp.zeros((1, CP), jnp.float32).at[:, :C].set(b1.reshape(1, C))
    return pl.pallas_call(
        _enc_body,
        out_shape=jax.ShapeDtypeStruct((H * W, CP), jnp.float32),
    )(patches, wf, b2d)


def _track_body(keymap_hbm, kp_hbm, keys_hbm, offs_hbm, w2_hbm,
                out_kp_hbm, out_keys_hbm,
                idx_ref, rows_ref, kp_v, keys_v, okp_v, okeys_v, offs_v,
                w2_v, sem):
    wid = lax.axis_index("s") * 2 + lax.axis_index("c")
    base_kp = wid * KP_PER_WORKER

    c1 = pltpu.async_copy(kp_hbm.at[pl.ds(wid * 16, 16)], kp_v, sem)
    c2 = pltpu.async_copy(keys_hbm.at[pl.ds(base_kp, KP_PER_WORKER)], keys_v, sem)
    c3 = pltpu.async_copy(offs_hbm, offs_v, sem)
    c4 = pltpu.async_copy(w2_hbm, w2_v, sem)
    c1.wait()
    c2.wait()
    c3.wait()
    c4.wait()

    off0 = offs_v[pl.ds(0, 16)]
    off1 = offs_v[pl.ds(16, 16)]
    kvec = kp_v[pl.ds(0, 16)]

    xs, ys, cxs, cys = [], [], [], []
    for j in range(KP_PER_WORKER):
        x = kvec[2 * j]
        y = kvec[2 * j + 1]
        cx = jnp.clip(x, 2, W - 3)
        cy = jnp.clip(y, 2, H - 3)
        xs.append(x); ys.append(y); cxs.append(cx); cys.append(cy)
        base = (cy - 2) * W + (cx - 2)
        idx_ref[j // 4, pl.ds((j % 4) * 32, 16)] = base + off0
        idx_ref[j // 4, pl.ds((j % 4) * 32 + 16, 16)] = base + off1

    cp0 = pltpu.async_copy(keymap_hbm.at[idx_ref.at[0]],
                           rows_ref.at[pl.ds(0, 128)], sem)
    cp1 = pltpu.async_copy(keymap_hbm.at[idx_ref.at[1]],
                           rows_ref.at[pl.ds(128, 128)], sem)
    cp0.wait()
    cp1.wait()

    lane = lax.iota(jnp.int32, 16)
    w2a = w2_v[pl.ds(0, 16)]
    w2b = w2_v[pl.ds(16, 16)]
    okp_vec = jnp.zeros((16,), jnp.int32)
    zf = jnp.zeros((16,), jnp.float32)
    for j in range(KP_PER_WORKER):
        k0 = keys_v[j, pl.ds(0, 16)]
        k1 = keys_v[j, pl.ds(16, 16)]
        rvec0 = lane + (j * 32)
        rvec1 = jnp.minimum(lane + 16, 24) + (j * 32)
        acc0 = zf
        acc1 = zf
        for c in range(C):
            kc = k0[c] if c < 16 else k1[c - 16]
            cvec = jnp.full((16,), c, jnp.int32)
            g0 = plsc.load_gather(rows_ref, [rvec0, cvec]) - kc
            g1 = plsc.load_gather(rows_ref, [rvec1, cvec]) - kc
            acc0 = acc0 + g0 * g0
            acc1 = acc1 + g1 * g1
        wd0 = acc0 * w2a
        wd1 = acc1 * w2b
        m0 = jnp.min(wd0)
        m1 = jnp.min(wd1)
        f0 = plsc.all_reduce_ffs(wd0 == m0)
        f1 = plsc.all_reduce_ffs(wd1 == m1)
        if hasattr(f0, 'ndim') and f0.ndim:
            f0 = f0[0]
            f1 = f1[0]
        in0 = m0 <= m1
        mi = jnp.where(in0, f0, f1 + 16)
        one = jnp.int32(1)
        zero = jnp.int32(0)
        min_y = (jnp.where(mi >= 5, one, zero) + jnp.where(mi >= 10, one, zero)
                 + jnp.where(mi >= 15, one, zero) + jnp.where(mi >= 20, one, zero))
        min_x = mi - 5 * min_y
        x, y, cx, cy = xs[j], ys[j], cxs[j], cys[j]
        tx = x - (cx - 2)
        ty = y - (cy - 2)
        nx = jnp.clip(x + tx - min_x, 0, W - 1)
        ny = jnp.clip(y + ty - min_y, 0, H - 1)
        px = nx - (cx - 2)
        py = ny - (cy - 2)
        rloc = j * 32 + py * 5 + px
        okeys_v[j, pl.ds(0, 16)] = rows_ref[rloc, pl.ds(0, 16)]
        okeys_v[j, pl.ds(16, 16)] = rows_ref[rloc, pl.ds(16, 16)]
        okp_vec = jnp.where(lane == 2 * j, nx, okp_vec)
        okp_vec = jnp.where(lane == 2 * j + 1, ny, okp_vec)

    okp_v[pl.ds(0, 16)] = okp_vec
    o1 = pltpu.async_copy(okp_v, out_kp_hbm.at[pl.ds(wid * 16, 16)], sem)
    o2 = pltpu.async_copy(okeys_v, out_keys_hbm.at[pl.ds(base_kp, KP_PER_WORKER)], sem)
    o1.wait()
    o2.wait()


def _tracker(keymap, memory_keypoints, memory_keys):
    mesh = plsc.VectorSubcoreMesh(core_axis_name="c", subcore_axis_name="s")
    run = functools.partial(
        pl.kernel,
        mesh=mesh,
        compiler_params=pltpu.CompilerParams(needs_layout_passes=False),
        out_type=[
            jax.ShapeDtypeStruct((K_POINTS * 2,), jnp.int32),
            jax.ShapeDtypeStruct((K_POINTS, C), jnp.float32),
        ],
        scratch_types=[
            pltpu.VMEM((2, 128), jnp.int32),
            pltpu.VMEM((KP_PER_WORKER * 32, CP), jnp.float32),
            pltpu.VMEM((KP_PER_WORKER * 2,), jnp.int32),
            pltpu.VMEM((KP_PER_WORKER, C), jnp.float32),
            pltpu.VMEM((KP_PER_WORKER * 2,), jnp.int32),
            pltpu.VMEM((KP_PER_WORKER, C), jnp.float32),
            pltpu.VMEM((32,), jnp.int32),
            pltpu.VMEM((32,), jnp.float32),
            pltpu.SemaphoreType.DMA,
        ],
    )(_track_body)
    offs = [(p // 5) * W + (p % 5) for p in range(25)]
    offs_const = jnp.asarray(np.array(offs + [offs[24]] * 7, np.int32))
    w2_const = jnp.asarray(np.array(_W2 + [1e30] * 7, np.float32))
    kp_flat, new_keys = run(keymap, memory_keypoints.reshape(-1), memory_keys,
                            offs_const, w2_const)
    return kp_flat.reshape(K_POINTS, 2), new_keys


def kernel(query, W1, b1, memory_keys, memory_keypoints):
    keymap = _encoder_keymap(query, W1, b1)
    return _tracker(keymap, memory_keypoints, memory_keys)


# speedup vs baseline: 1.1148x; 1.1148x over previous
import functools

import jax
import jax.numpy as jnp
import numpy as np
from jax import lax
from jax.experimental import pallas as pl
from jax.experimental.pallas import tpu as pltpu
from jax.experimental.pallas import tpu_sc as plsc

K_POINTS = 256
H = W = 64
C = 32
CP = 128
NUM_WORKERS = 32
KP_PER_WORKER = K_POINTS // NUM_WORKERS

_ii = np.arange(5)
_cheb = np.maximum(np.abs(_ii - 2)[:, None], np.abs(_ii - 2)[None, :]).astype(np.float32)
_wmap = (np.float32(0.5) + np.float32(0.1) * _cheb).astype(np.float32)
_W2 = [float(np.float32(w) * np.float32(w)) for w in _wmap.reshape(-1)]


def _enc_body(q_ref, w_ref, b_ref, o_ref):
    q = q_ref[...]
    p = q.reshape(3, H, 4, W, 4).transpose(1, 3, 0, 2, 4).reshape(H * W, 48)
    y = jnp.dot(p, w_ref[...], preferred_element_type=jnp.float32)
    o_ref[...] = jnp.maximum(y + b_ref[...], 0.0)


def _encoder_keymap(query, W1, b1):
    patches = query[0].reshape(3 * 256, 256)
    wf = jnp.zeros((48, CP), jnp.float32).at[:, :C].set(W1.reshape(C, 48).T)
    b2d = jnp.zeros((1, CP), jnp.float32).at[:, :C].set(b1.reshape(1, C))
    return pl.pallas_call(
        _enc_body,
        out_shape=jax.ShapeDtypeStruct((H * W, CP), jnp.float32),
    )(patches, wf, b2d)


def _track_body(keymap_hbm, kp_hbm, keys_hbm, offs_hbm, w2_hbm,
                out_kp_hbm, out_keys_hbm,
                idx_ref, rows_ref, kp_v, keys_v, okp_v, okeys_v, offs_v,
                w2_v, sem):
    wid = lax.axis_index("s") * 2 + lax.axis_index("c")
    base_kp = wid * KP_PER_WORKER

    c1 = pltpu.async_copy(kp_hbm.at[pl.ds(wid * 16, 16)], kp_v, sem)
    c2 = pltpu.async_copy(keys_hbm.at[pl.ds(base_kp, KP_PER_WORKER)], keys_v, sem)
    c3 = pltpu.async_copy(offs_hbm, offs_v, sem)
    c4 = pltpu.async_copy(w2_hbm, w2_v, sem)
    c1.wait()
    c2.wait()
    c3.wait()
    c4.wait()

    off0 = offs_v[pl.ds(0, 16)]
    off1 = offs_v[pl.ds(16, 16)]
    kvec = kp_v[pl.ds(0, 16)]

    xs, ys, cxs, cys = [], [], [], []
    for j in range(KP_PER_WORKER):
        x = kvec[2 * j]
        y = kvec[2 * j + 1]
        cx = jnp.clip(x, 2, W - 3)
        cy = jnp.clip(y, 2, H - 3)
        xs.append(x); ys.append(y); cxs.append(cx); cys.append(cy)
        base = (cy - 2) * W + (cx - 2)
        idx_ref[j // 4, pl.ds((j % 4) * 32, 16)] = base + off0
        idx_ref[j // 4, pl.ds((j % 4) * 32 + 16, 16)] = base + off1

    cp0 = pltpu.async_copy(keymap_hbm.at[idx_ref.at[0]],
                           rows_ref.at[pl.ds(0, 128)], sem)
    cp1 = pltpu.async_copy(keymap_hbm.at[idx_ref.at[1]],
                           rows_ref.at[pl.ds(128, 128)], sem)
    cp0.wait()
    cp1.wait()

    lane = lax.iota(jnp.int32, 16)
    okp_vec = jnp.zeros((16,), jnp.int32)
    for j in range(KP_PER_WORKER):
        k0 = keys_v[j, pl.ds(0, 16)]
        k1 = keys_v[j, pl.ds(16, 16)]
        m = None
        mi = None
        for p in range(25):
            r0 = rows_ref[j * 32 + p, pl.ds(0, 16)]
            r1 = rows_ref[j * 32 + p, pl.ds(16, 16)]
            d0 = r0 - k0
            d1 = r1 - k1
            wd = jnp.sum(d0 * d0 + d1 * d1) * jnp.float32(_W2[p])
            if m is None:
                m = wd
                mi = jnp.int32(0)
            else:
                pred = wd < m
                mi = jnp.where(pred, jnp.int32(p), mi)
                m = jnp.where(pred, wd, m)
        one = jnp.int32(1)
        zero = jnp.int32(0)
        min_y = (jnp.where(mi >= 5, one, zero) + jnp.where(mi >= 10, one, zero)
                 + jnp.where(mi >= 15, one, zero) + jnp.where(mi >= 20, one, zero))
        min_x = mi - 5 * min_y
        x, y, cx, cy = xs[j], ys[j], cxs[j], cys[j]
        tx = x - (cx - 2)
        ty = y - (cy - 2)
        nx = jnp.clip(x + tx - min_x, 0, W - 1)
        ny = jnp.clip(y + ty - min_y, 0, H - 1)
        px = nx - (cx - 2)
        py = ny - (cy - 2)
        rloc = j * 32 + py * 5 + px
        okeys_v[j, pl.ds(0, 16)] = rows_ref[rloc, pl.ds(0, 16)]
        okeys_v[j, pl.ds(16, 16)] = rows_ref[rloc, pl.ds(16, 16)]
        okp_vec = jnp.where(lane == 2 * j, nx, okp_vec)
        okp_vec = jnp.where(lane == 2 * j + 1, ny, okp_vec)

    okp_v[pl.ds(0, 16)] = okp_vec
    o1 = pltpu.async_copy(okp_v, out_kp_hbm.at[pl.ds(wid * 16, 16)], sem)
    o2 = pltpu.async_copy(okeys_v, out_keys_hbm.at[pl.ds(base_kp, KP_PER_WORKER)], sem)
    o1.wait()
    o2.wait()


def _tracker(keymap, memory_keypoints, memory_keys):
    mesh = plsc.VectorSubcoreMesh(core_axis_name="c", subcore_axis_name="s")
    run = functools.partial(
        pl.kernel,
        mesh=mesh,
        compiler_params=pltpu.CompilerParams(needs_layout_passes=False),
        out_type=[
            jax.ShapeDtypeStruct((K_POINTS * 2,), jnp.int32),
            jax.ShapeDtypeStruct((K_POINTS, C), jnp.float32),
        ],
        scratch_types=[
            pltpu.VMEM((2, 128), jnp.int32),
            pltpu.VMEM((KP_PER_WORKER * 32, CP), jnp.float32),
            pltpu.VMEM((KP_PER_WORKER * 2,), jnp.int32),
            pltpu.VMEM((KP_PER_WORKER, C), jnp.float32),
            pltpu.VMEM((KP_PER_WORKER * 2,), jnp.int32),
            pltpu.VMEM((KP_PER_WORKER, C), jnp.float32),
            pltpu.VMEM((32,), jnp.int32),
            pltpu.VMEM((32,), jnp.float32),
            pltpu.SemaphoreType.DMA,
        ],
    )(_track_body)
    offs = [(p // 5) * W + (p % 5) for p in range(25)]
    offs_const = jnp.asarray(np.array(offs + [offs[24]] * 7, np.int32))
    w2_const = jnp.asarray(np.array(_W2 + [1e30] * 7, np.float32))
    kp_flat, new_keys = run(keymap, memory_keypoints.reshape(-1), memory_keys,
                            offs_const, w2_const)
    return kp_flat.reshape(K_POINTS, 2), new_keys


def kernel(query, W1, b1, memory_keys, memory_keypoints):
    keymap = _encoder_keymap(query, W1, b1)
    return _tracker(keymap, memory_keypoints, memory_keys)


# speedup vs baseline: 1.2253x; 1.0991x over previous
import functools

import jax
import jax.numpy as jnp
import numpy as np
from jax import lax
from jax.experimental import pallas as pl
from jax.experimental.pallas import tpu as pltpu
from jax.experimental.pallas import tpu_sc as plsc

K_POINTS = 256
H = W = 64
C = 32
CP = 128
NUM_WORKERS = 32
KP_PER_WORKER = K_POINTS // NUM_WORKERS

_ii = np.arange(5)
_cheb = np.maximum(np.abs(_ii - 2)[:, None], np.abs(_ii - 2)[None, :]).astype(np.float32)
_wmap = (np.float32(0.5) + np.float32(0.1) * _cheb).astype(np.float32)
_W2 = [float(np.float32(w) * np.float32(w)) for w in _wmap.reshape(-1)]


def _enc_body(q_ref, w_ref, b_ref, o_ref):
    q = q_ref[...].reshape(3 * 256, 256)
    w = w_ref[...].reshape(C, 48)
    acc = None
    for c in range(3):
        qc = lax.slice(q, (c * 256, 0), ((c + 1) * 256, 256))
        p16 = qc.reshape(H, 4, W, 4).transpose(0, 2, 1, 3).reshape(H * W, 16)
        wc = lax.slice(w, (0, c * 16), (C, (c + 1) * 16))
        part = lax.dot_general(p16, wc, (((1,), (1,)), ((), ())),
                               preferred_element_type=jnp.float32)
        acc = part if acc is None else acc + part
    o_ref[:, pl.ds(0, C)] = jnp.maximum(acc + b_ref[...], 0.0)


def _encoder_keymap(query, W1, b1):
    return pl.pallas_call(
        _enc_body,
        out_shape=jax.ShapeDtypeStruct((H * W, CP), jnp.float32),
    )(query, W1, b1.reshape(1, C))


def _track_body(keymap_hbm, kp_hbm, keys_hbm, offs_hbm, w2_hbm,
                out_kp_hbm, out_keys_hbm,
                idx_ref, rows_ref, kp_v, keys_v, okp_v, okeys_v, offs_v,
                w2_v, sem):
    wid = lax.axis_index("s") * 2 + lax.axis_index("c")
    base_kp = wid * KP_PER_WORKER

    c1 = pltpu.async_copy(kp_hbm.at[pl.ds(wid * 16, 16)], kp_v, sem)
    c2 = pltpu.async_copy(keys_hbm.at[pl.ds(base_kp, KP_PER_WORKER)], keys_v, sem)
    c3 = pltpu.async_copy(offs_hbm, offs_v, sem)
    c4 = pltpu.async_copy(w2_hbm, w2_v, sem)
    c1.wait()
    c2.wait()
    c3.wait()
    c4.wait()

    off0 = offs_v[pl.ds(0, 16)]
    off1 = offs_v[pl.ds(16, 16)]
    kvec = kp_v[pl.ds(0, 16)]

    xs, ys, cxs, cys = [], [], [], []
    for j in range(KP_PER_WORKER):
        x = kvec[2 * j]
        y = kvec[2 * j + 1]
        cx = jnp.clip(x, 2, W - 3)
        cy = jnp.clip(y, 2, H - 3)
        xs.append(x); ys.append(y); cxs.append(cx); cys.append(cy)
        base = (cy - 2) * W + (cx - 2)
        idx_ref[j // 4, pl.ds((j % 4) * 32, 16)] = base + off0
        idx_ref[j // 4, pl.ds((j % 4) * 32 + 16, 16)] = base + off1

    cp0 = pltpu.async_copy(keymap_hbm.at[idx_ref.at[0]],
                           rows_ref.at[pl.ds(0, 128)], sem)
    cp1 = pltpu.async_copy(keymap_hbm.at[idx_ref.at[1]],
                           rows_ref.at[pl.ds(128, 128)], sem)
    cp0.wait()
    cp1.wait()

    lane = lax.iota(jnp.int32, 16)
    okp_vec = jnp.zeros((16,), jnp.int32)
    for j in range(KP_PER_WORKER):
        k0 = keys_v[j, pl.ds(0, 16)]
        k1 = keys_v[j, pl.ds(16, 16)]
        m = None
        mi = None
        for p in range(25):
            r0 = rows_ref[j * 32 + p, pl.ds(0, 16)]
            r1 = rows_ref[j * 32 + p, pl.ds(16, 16)]
            d0 = r0 - k0
            d1 = r1 - k1
            wd = jnp.sum(d0 * d0 + d1 * d1) * jnp.float32(_W2[p])
            if m is None:
                m = wd
                mi = jnp.int32(0)
            else:
                pred = wd < m
                mi = jnp.where(pred, jnp.int32(p), mi)
                m = jnp.where(pred, wd, m)
        one = jnp.int32(1)
        zero = jnp.int32(0)
        min_y = (jnp.where(mi >= 5, one, zero) + jnp.where(mi >= 10, one, zero)
                 + jnp.where(mi >= 15, one, zero) + jnp.where(mi >= 20, one, zero))
        min_x = mi - 5 * min_y
        x, y, cx, cy = xs[j], ys[j], cxs[j], cys[j]
        tx = x - (cx - 2)
        ty = y - (cy - 2)
        nx = jnp.clip(x + tx - min_x, 0, W - 1)
        ny = jnp.clip(y + ty - min_y, 0, H - 1)
        px = nx - (cx - 2)
        py = ny - (cy - 2)
        rloc = j * 32 + py * 5 + px
        okeys_v[j, pl.ds(0, 16)] = rows_ref[rloc, pl.ds(0, 16)]
        okeys_v[j, pl.ds(16, 16)] = rows_ref[rloc, pl.ds(16, 16)]
        okp_vec = jnp.where(lane == 2 * j, nx, okp_vec)
        okp_vec = jnp.where(lane == 2 * j + 1, ny, okp_vec)

    okp_v[pl.ds(0, 16)] = okp_vec
    o1 = pltpu.async_copy(okp_v, out_kp_hbm.at[pl.ds(wid * 16, 16)], sem)
    o2 = pltpu.async_copy(okeys_v, out_keys_hbm.at[pl.ds(base_kp, KP_PER_WORKER)], sem)
    o1.wait()
    o2.wait()


def _tracker(keymap, memory_keypoints, memory_keys):
    mesh = plsc.VectorSubcoreMesh(core_axis_name="c", subcore_axis_name="s")
    run = functools.partial(
        pl.kernel,
        mesh=mesh,
        compiler_params=pltpu.CompilerParams(needs_layout_passes=False),
        out_type=[
            jax.ShapeDtypeStruct((K_POINTS * 2,), jnp.int32),
            jax.ShapeDtypeStruct((K_POINTS, C), jnp.float32),
        ],
        scratch_types=[
            pltpu.VMEM((2, 128), jnp.int32),
            pltpu.VMEM((KP_PER_WORKER * 32, CP), jnp.float32),
            pltpu.VMEM((KP_PER_WORKER * 2,), jnp.int32),
            pltpu.VMEM((KP_PER_WORKER, C), jnp.float32),
            pltpu.VMEM((KP_PER_WORKER * 2,), jnp.int32),
            pltpu.VMEM((KP_PER_WORKER, C), jnp.float32),
            pltpu.VMEM((32,), jnp.int32),
            pltpu.VMEM((32,), jnp.float32),
            pltpu.SemaphoreType.DMA,
        ],
    )(_track_body)
    offs = [(p // 5) * W + (p % 5) for p in range(25)]
    offs_const = jnp.asarray(np.array(offs + [offs[24]] * 7, np.int32))
    w2_const = jnp.asarray(np.array(_W2 + [1e30] * 7, np.float32))
    kp_flat, new_keys = run(keymap, memory_keypoints.reshape(-1), memory_keys,
                            offs_const, w2_const)
    return kp_flat.reshape(K_POINTS, 2), new_keys


def kernel(query, W1, b1, memory_keys, memory_keypoints):
    keymap = _encoder_keymap(query, W1, b1)
    return _tracker(keymap, memory_keypoints, memory_keys)


# speedup vs baseline: 1.2396x; 1.0117x over previous
import functools

import jax
import jax.numpy as jnp
import numpy as np
from jax import lax
from jax.experimental import pallas as pl
from jax.experimental.pallas import tpu as pltpu
from jax.experimental.pallas import tpu_sc as plsc

K_POINTS = 256
H = W = 64
C = 32
CP = 128
NUM_WORKERS = 32
KP_PER_WORKER = K_POINTS // NUM_WORKERS

_ii = np.arange(5)
_cheb = np.maximum(np.abs(_ii - 2)[:, None], np.abs(_ii - 2)[None, :]).astype(np.float32)
_wmap = (np.float32(0.5) + np.float32(0.1) * _cheb).astype(np.float32)
_W2 = [float(np.float32(w) * np.float32(w)) for w in _wmap.reshape(-1)]


def _enc_body(q_ref, w_ref, b_ref, o_ref):
    q = q_ref[...].reshape(3 * 256, 256)
    w = w_ref[...].reshape(C, 48)
    acc = None
    for c in range(3):
        qc = lax.slice(q, (c * 256, 0), ((c + 1) * 256, 256))
        p16 = qc.reshape(H, 4, W, 4).transpose(0, 2, 1, 3).reshape(H * W, 16)
        wc = lax.slice(w, (0, c * 16), (C, (c + 1) * 16))
        part = lax.dot_general(p16, wc, (((1,), (1,)), ((), ())),
                               preferred_element_type=jnp.float32)
        acc = part if acc is None else acc + part
    o_ref[:, pl.ds(0, C)] = jnp.maximum(acc + b_ref[...], 0.0)


def _encoder_keymap(query, W1, b1):
    return pl.pallas_call(
        _enc_body,
        out_shape=jax.ShapeDtypeStruct((H * W, CP), jnp.float32),
    )(query, W1, b1.reshape(1, C))


def _track_body(keymap_hbm, kp_hbm, keys_hbm, offs_hbm,
                out_kp_hbm, out_keys_hbm,
                idx_ref, rows_ref, kp_v, keys_v, okp_v, okeys_v, offs_v,
                sem):
    wid = lax.axis_index("s") * 2 + lax.axis_index("c")
    base_kp = wid * KP_PER_WORKER

    c1 = pltpu.async_copy(kp_hbm.at[pl.ds(wid * 16, 16)], kp_v, sem)
    c2 = pltpu.async_copy(keys_hbm.at[pl.ds(base_kp, KP_PER_WORKER)], keys_v, sem)
    c3 = pltpu.async_copy(offs_hbm, offs_v, sem)
    c1.wait()
    c2.wait()
    c3.wait()

    off0 = offs_v[pl.ds(0, 16)]
    off1 = offs_v[pl.ds(16, 16)]
    kvec = kp_v[pl.ds(0, 16)]

    xs, ys, cxs, cys = [], [], [], []
    for j in range(KP_PER_WORKER):
        x = kvec[2 * j]
        y = kvec[2 * j + 1]
        cx = jnp.clip(x, 2, W - 3)
        cy = jnp.clip(y, 2, H - 3)
        xs.append(x); ys.append(y); cxs.append(cx); cys.append(cy)
        base = (cy - 2) * W + (cx - 2)
        idx_ref[j // 4, pl.ds((j % 4) * 32, 16)] = base + off0
        idx_ref[j // 4, pl.ds((j % 4) * 32 + 16, 16)] = base + off1

    cp0 = pltpu.async_copy(keymap_hbm.at[idx_ref.at[0]],
                           rows_ref.at[pl.ds(0, 128)], sem)
    cp1 = pltpu.async_copy(keymap_hbm.at[idx_ref.at[1]],
                           rows_ref.at[pl.ds(128, 128)], sem)
    cp0.wait()
    cp1.wait()

    lane = lax.iota(jnp.int32, 16)
    okp_vec = jnp.zeros((16,), jnp.int32)
    for j in range(KP_PER_WORKER):
        k0 = keys_v[j, pl.ds(0, 16)]
        k1 = keys_v[j, pl.ds(16, 16)]
        m = None
        mi = None
        for p in range(25):
            r0 = rows_ref[j * 32 + p, pl.ds(0, 16)]
            r1 = rows_ref[j * 32 + p, pl.ds(16, 16)]
            d0 = r0 - k0
            d1 = r1 - k1
            wd = jnp.sum(d0 * d0 + d1 * d1) * jnp.float32(_W2[p])
            if m is None:
                m = wd
                mi = jnp.int32(0)
            else:
                pred = wd < m
                mi = jnp.where(pred, jnp.int32(p), mi)
                m = jnp.where(pred, wd, m)
        one = jnp.int32(1)
        zero = jnp.int32(0)
        min_y = (jnp.where(mi >= 5, one, zero) + jnp.where(mi >= 10, one, zero)
                 + jnp.where(mi >= 15, one, zero) + jnp.where(mi >= 20, one, zero))
        min_x = mi - 5 * min_y
        x, y, cx, cy = xs[j], ys[j], cxs[j], cys[j]
        tx = x - (cx - 2)
        ty = y - (cy - 2)
        nx = jnp.clip(x + tx - min_x, 0, W - 1)
        ny = jnp.clip(y + ty - min_y, 0, H - 1)
        px = nx - (cx - 2)
        py = ny - (cy - 2)
        rloc = j * 32 + py * 5 + px
        okeys_v[j, pl.ds(0, 16)] = rows_ref[rloc, pl.ds(0, 16)]
        okeys_v[j, pl.ds(16, 16)] = rows_ref[rloc, pl.ds(16, 16)]
        okp_vec = jnp.where(lane == 2 * j, nx, okp_vec)
        okp_vec = jnp.where(lane == 2 * j + 1, ny, okp_vec)

    okp_v[pl.ds(0, 16)] = okp_vec
    o1 = pltpu.async_copy(okp_v, out_kp_hbm.at[pl.ds(wid * 16, 16)], sem)
    o2 = pltpu.async_copy(okeys_v, out_keys_hbm.at[pl.ds(base_kp, KP_PER_WORKER)], sem)
    o1.wait()
    o2.wait()


def _tracker(keymap, memory_keypoints, memory_keys):
    mesh = plsc.VectorSubcoreMesh(core_axis_name="c", subcore_axis_name="s")
    run = functools.partial(
        pl.kernel,
        mesh=mesh,
        compiler_params=pltpu.CompilerParams(needs_layout_passes=False),
        out_type=[
            jax.ShapeDtypeStruct((K_POINTS * 2,), jnp.int32),
            jax.ShapeDtypeStruct((K_POINTS, C), jnp.float32),
        ],
        scratch_types=[
            pltpu.VMEM((2, 128), jnp.int32),
            pltpu.VMEM((KP_PER_WORKER * 32, CP), jnp.float32),
            pltpu.VMEM((KP_PER_WORKER * 2,), jnp.int32),
            pltpu.VMEM((KP_PER_WORKER, C), jnp.float32),
            pltpu.VMEM((KP_PER_WORKER * 2,), jnp.int32),
            pltpu.VMEM((KP_PER_WORKER, C), jnp.float32),
            pltpu.VMEM((32,), jnp.int32),
            pltpu.SemaphoreType.DMA,
        ],
    )(_track_body)
    offs = [(p // 5) * W + (p % 5) for p in range(25)]
    offs_const = jnp.asarray(np.array(offs + [offs[24]] * 7, np.int32))
    kp_flat, new_keys = run(keymap, memory_keypoints.reshape(-1), memory_keys,
                            offs_const)
    return kp_flat.reshape(K_POINTS, 2), new_keys


def kernel(query, W1, b1, memory_keys, memory_keypoints):
    keymap = _encoder_keymap(query, W1, b1)
    return _tracker(keymap, memory_keypoints, memory_keys)
